# baseline (speedup 1.0000x reference)
import functools

import jax
import jax.numpy as jnp
import numpy as np
from jax import lax
from jax.experimental import pallas as pl
from jax.experimental.pallas import tpu as pltpu
from jax.experimental.pallas import tpu_sc as plsc

NC = 2
NS = 16
LANES = 16
DH = 128
CHUNK = 80


def _bf16_perm():
    a = np.empty(2 * DH // 2, np.int32)
    for j in range(2 * DH // 2):
        half, p = divmod(j, DH // 2)
        k, i = divmod(p, 16)
        a[j] = DH * half + 32 * k + i
    return np.concatenate([a, a + 16])


def _mp_sc(n_nodes: int, n_edges: int):
    ept = n_edges // NS
    nfull = ept // CHUNK
    ntrip = nfull // 3
    rest = ept - ntrip * 3 * CHUNK
    rpt_a = (-(-n_nodes // NS) + 7) // 8 * 8
    rpt_last = n_nodes - (NS - 1) * rpt_a

    mesh = plsc.VectorSubcoreMesh(
        core_axis_name="c", subcore_axis_name="s", num_cores=NC, num_subcores=NS
    )

    NB = 3
    scratch = (
        [pltpu.VMEM((CHUNK,), jnp.int32) for _ in range(NB)]
        + [pltpu.VMEM((CHUNK,), jnp.int32) for _ in range(NB)]
        + [pltpu.VMEM((CHUNK, DH), jnp.float32) for _ in range(NB)]
        + [pltpu.VMEM((CHUNK // 2, DH), jnp.int32) for _ in range(NB)]
        + [pltpu.VMEM_SHARED((n_nodes, DH), jnp.float32)]
        + [pltpu.SemaphoreType.DMA for _ in range(3 * NB)]
    )
    if rest:
        scratch += [pltpu.VMEM((rest,), jnp.int32),
                    pltpu.VMEM((rest,), jnp.int32)]

    @functools.partial(
        pl.kernel,
        out_type=jax.ShapeDtypeStruct((2 * n_nodes, DH), jnp.float32),
        mesh=mesh,
        scratch_types=scratch,
    )
    def mp(h2_hbm, e2_hbm, src_hbm, dst_hbm, agg_hbm, *scr):
        srcs = scr[0:NB]
        dsts = scr[NB:2 * NB]
        rows = scr[2 * NB:3 * NB]
        evs = scr[3 * NB:4 * NB]
        agg_sh = scr[4 * NB]
        semi = scr[4 * NB + 1:4 * NB + 1 + NB]
        semg = scr[4 * NB + 1 + NB:4 * NB + 1 + 2 * NB]
        sems = scr[4 * NB + 1 + 2 * NB:4 * NB + 1 + 3 * NB]
        if rest:
            src_t, dst_t = scr[-2], scr[-1]

        c = lax.axis_index("c")
        t = lax.axis_index("s")
        coff = c * n_nodes
        eoff = c * n_edges
        ebase = t * ept

        zeros = jnp.zeros((LANES,), jnp.float32)

        @pl.loop(0, CHUNK)
        def _zero_fill(i):
            for k in range(DH // LANES):
                rows[0][i, pl.ds(k * LANES, LANES)] = zeros

        r0 = t * rpt_a

        def _zero_rows(nrows):
            nf = nrows // CHUNK
            rem = nrows - nf * CHUNK

            @pl.loop(0, nf)
            def _z(i):
                pltpu.sync_copy(rows[0], agg_sh.at[pl.ds(r0 + i * CHUNK, CHUNK)])
            if rem:
                pltpu.sync_copy(rows[0].at[pl.ds(0, rem)],
                                agg_sh.at[pl.ds(r0 + nf * CHUNK, rem)])

        @pl.when(t < NS - 1)
        def _zero_main():
            _zero_rows(rpt_a)

        @pl.when(t == NS - 1)
        def _zero_last():
            _zero_rows(rpt_last)

        plsc.subcore_barrier()

        def issue_idx(chunk_i, b):
            base = ebase + chunk_i * CHUNK
            pltpu.async_copy(src_hbm.at[pl.ds(base, CHUNK)], srcs[b], semi[b])
            pltpu.async_copy(dst_hbm.at[pl.ds(base, CHUNK)], dsts[b], semi[b])

        def wait_idx(b):
            pltpu.make_async_copy(src_hbm.at[pl.ds(0, CHUNK)], srcs[b],
                                  semi[b]).wait()
            pltpu.make_async_copy(dst_hbm.at[pl.ds(0, CHUNK)], dsts[b],
                                  semi[b]).wait()

        def issue_gather(chunk_i, b):
            for k in range(CHUNK // LANES):
                sl = pl.ds(k * LANES, LANES)
                srcs[b][sl] = srcs[b][sl] + coff
            pltpu.async_copy(h2_hbm.at[srcs[b]], rows[b], semg[b])
            erow = pl.multiple_of(
                eoff // 2 + (t * (ept // 2)) + chunk_i * (CHUNK // 2), 8)
            pltpu.async_copy(e2_hbm.at[pl.ds(erow, CHUNK // 2)],
                             evs[b], semg[b])

        def wait_gather(b):
            pltpu.make_async_copy(h2_hbm.at[srcs[b]], rows[b], semg[b]).wait()
            pltpu.make_async_copy(e2_hbm.at[pl.ds(0, CHUNK // 2)], evs[b],
                                  semg[b]).wait()

        def _combine_pair(b, p):
            for q in range(2):
                r = 2 * p + q
                for k in range(DH // (2 * LANES)):
                    w16 = evs[b][p, pl.ds(q * (DH // 2) + LANES * k, LANES)]
                    ea = lax.bitcast_convert_type(lax.shift_left(w16, 16),
                                                  jnp.float32)
                    eb = lax.bitcast_convert_type(w16 & jnp.int32(-65536),
                                                  jnp.float32)
                    sla = pl.ds(2 * LANES * k, LANES)
                    slb = pl.ds(2 * LANES * k + LANES, LANES)
                    rows[b][r, sla] = jnp.maximum(rows[b][r, sla] + ea, 0.0)
                    rows[b][r, slb] = jnp.maximum(rows[b][r, slb] + eb, 0.0)

        def relu_combine(b):
            @pl.loop(0, CHUNK // 2)
            def _row(p):
                _combine_pair(b, p)

        def issue_scatter(b):
            pltpu.async_copy(rows[b], agg_sh.at[dsts[b]], sems[b], add=True)

        def wait_scatter(b):
            pltpu.make_async_copy(rows[b], agg_sh.at[dsts[b]], sems[b]).wait()

        if ntrip > 0:
            issue_idx(0, 0)
            issue_idx(1, 1)
            issue_idx(2, 2)
            wait_idx(0)
            issue_gather(0, 0)
            wait_idx(1)
            issue_gather(1, 1)

            @pl.loop(0, ntrip)
            def _triple(tr):
                a = 3 * tr

                @pl.when(tr > 0)
                def _refill_c():
                    wait_scatter(2)
                    issue_idx(a + 2, 2)

                wait_idx(2)
                issue_gather(a + 2, 2)

                wait_gather(0)
                relu_combine(0)
                issue_scatter(0)
                wait_gather(1)
                relu_combine(1)
                issue_scatter(1)

                @pl.when(tr < ntrip - 1)
                def _refill_a():
                    wait_scatter(0)
                    issue_idx(a + 3, 0)

                wait_gather(2)
                relu_combine(2)
                issue_scatter(2)

                @pl.when(tr < ntrip - 1)
                def _refill_b():
                    wait_scatter(1)
                    issue_idx(a + 4, 1)

                @pl.when(tr < ntrip - 1)
                def _regather():
                    wait_idx(0)
                    issue_gather(a + 3, 0)
                    wait_idx(1)
                    issue_gather(a + 4, 1)

            wait_scatter(0)
            wait_scatter(1)
            wait_scatter(2)

        if rest:
            cc = rest
            base = ebase + ntrip * 3 * CHUNK
            pltpu.sync_copy(src_hbm.at[pl.ds(base, cc)], src_t)
            pltpu.sync_copy(dst_hbm.at[pl.ds(base, cc)], dst_t)
            for k in range(cc // LANES):
                sl = pl.ds(k * LANES, LANES)
                src_t[sl] = src_t[sl] + coff
            g = pltpu.async_copy(h2_hbm.at[src_t],
                                 rows[0].at[pl.ds(0, cc)], semg[0])
            erow = pl.multiple_of(eoff // 2 + base // 2, 8)
            le = pltpu.async_copy(e2_hbm.at[pl.ds(erow, cc // 2)],
                                  evs[0].at[pl.ds(0, cc // 2)], semg[0])
            g.wait()
            le.wait()

            @pl.loop(0, cc // 2)
            def _rowt(p):
                _combine_pair(0, p)

            pltpu.sync_copy(rows[0].at[pl.ds(0, cc)],
                            agg_sh.at[dst_t], add=True)

        plsc.subcore_barrier()

        @pl.when(t < NS - 1)
        def _flush_main():
            pltpu.sync_copy(agg_sh.at[pl.ds(r0, rpt_a)],
                            agg_hbm.at[pl.ds(coff + r0, rpt_a)])

        @pl.when(t == NS - 1)
        def _flush_last():
            pltpu.sync_copy(agg_sh.at[pl.ds(r0, rpt_last)],
                            agg_hbm.at[pl.ds(coff + r0, rpt_last)])

    return mp


def _enc_h(x, w, b):
    n = x.shape[0]

    def body(x_ref, w_ref, b_ref, o_ref):
        h = jnp.dot(x_ref[...], w_ref[...], preferred_element_type=jnp.float32)
        h = h + b_ref[...]
        o_ref[0] = h[:, :DH]
        o_ref[1] = h[:, DH:]

    return pl.pallas_call(
        body,
        out_shape=jax.ShapeDtypeStruct((2, n, DH), jnp.float32),
    )(x, w, b)


def _enc_e(edge_attr, w, b, block_e=4000):
    e_cnt, dk = edge_attr.shape
    grid = (e_cnt // block_e,)

    def body(a_ref, w_ref, b_ref, o_ref):
        eb = jnp.dot(a_ref[...], w_ref[...], preferred_element_type=jnp.float32)
        eb = eb + b_ref[...]
        u = lax.bitcast_convert_type(eb, jnp.uint32) + jnp.uint32(0x8000)
        lo = u[:, :DH] >> 16
        hi = u[:, DH:] & jnp.uint32(0xFFFF0000)
        words = lax.bitcast_convert_type(hi | lo, jnp.int32)
        o_ref[0] = words[:, :DH // 2]
        o_ref[1] = words[:, DH // 2:]

    return pl.pallas_call(
        body,
        grid=grid,
        in_specs=[
            pl.BlockSpec((block_e, dk), lambda i: (i, 0)),
            pl.BlockSpec((dk, 2 * DH), lambda i: (0, 0)),
            pl.BlockSpec((1, 2 * DH), lambda i: (0, 0)),
        ],
        out_specs=pl.BlockSpec((2, block_e, DH // 2), lambda i: (0, i, 0)),
        out_shape=jax.ShapeDtypeStruct((2, e_cnt, DH // 2), jnp.int32),
    )(edge_attr, w, b)


def _layer_update(h2, agg2, w, b, gamma, beta):
    n = h2.shape[1]

    def body(h_ref, a_ref, w_ref, b_ref, g_ref, bt_ref, o_ref):
        s0 = h_ref[0] + a_ref[0]
        s1 = h_ref[1] + a_ref[1]
        z = jnp.dot(s0, w_ref[:DH], preferred_element_type=jnp.float32)
        z = z + jnp.dot(s1, w_ref[DH:], preferred_element_type=jnp.float32)
        z = z + b_ref[...]
        mean = jnp.mean(z, axis=0, keepdims=True)
        zc = z - mean
        var = jnp.mean(zc * zc, axis=0, keepdims=True)
        zn = zc / jnp.sqrt(var + 1e-5) * g_ref[...] + bt_ref[...]
        r = jnp.maximum(zn, 0.0)
        o_ref[0] = r[:, :DH] + h_ref[0]
        o_ref[1] = r[:, DH:] + h_ref[1]

    return pl.pallas_call(
        body,
        out_shape=jax.ShapeDtypeStruct((2, n, DH), jnp.float32),
    )(h2, agg2, w, b, gamma, beta)


def _pool_decode(h2, batch2, wd1, bd1, wd2, bd2, n_graphs=64):
    n = h2.shape[1]
    nout = wd2.shape[1]

    def body(h_ref, b_ref, w1_ref, b1_ref, w2_ref, b2_ref, o_ref):
        gids = lax.broadcasted_iota(jnp.int32, (n_graphs, n), 0)
        onehot = (gids == b_ref[...]).astype(jnp.float32)
        sums0 = jnp.dot(onehot, h_ref[0], preferred_element_type=jnp.float32)
        sums1 = jnp.dot(onehot, h_ref[1], preferred_element_type=jnp.float32)
        counts = jnp.sum(onehot, axis=1, keepdims=True)
        cmax = jnp.maximum(counts, 1.0)
        p0 = sums0 / cmax
        p1 = sums1 / cmax
        r = jnp.dot(p0, w1_ref[:DH], preferred_element_type=jnp.float32)
        r = r + jnp.dot(p1, w1_ref[DH:], preferred_element_type=jnp.float32)
        r = jnp.maximum(r + b1_ref[...], 0.0)
        o_ref[...] = jnp.dot(r, w2_ref[...], preferred_element_type=jnp.float32) + b2_ref[...]

    return pl.pallas_call(
        body,
        out_shape=jax.ShapeDtypeStruct((n_graphs, nout), jnp.float32),
    )(h2, batch2, wd1, bd1, wd2, bd2)


def kernel(x, edge_index, edge_attr, batch, W_in, b_in, W_e, b_e,
           Ws, bs, gammas, betas, Wd1, bd1, Wd2, bd2):
    n = x.shape[0]
    e_cnt = edge_index.shape[1]
    nlayer = Ws.shape[0]
    n_graphs = 64

    src = edge_index[0]
    dst = edge_index[1]

    h2 = _enc_h(x, W_in, b_in.reshape(1, -1))
    perm = _bf16_perm()
    e2 = _enc_e(edge_attr, W_e[:, perm], b_e[perm].reshape(1, -1))
    e2f = e2.reshape(e_cnt, DH)

    mp = _mp_sc(n, e_cnt)
    for l in range(nlayer):
        agg = mp(h2.reshape(2 * n, DH), e2f, src, dst)
        h2 = _layer_update(h2, agg.reshape(2, n, DH), Ws[l],
                           bs[l].reshape(1, -1), gammas[l].reshape(1, -1),
                           betas[l].reshape(1, -1))

    return _pool_decode(h2, batch.reshape(1, -1), Wd1, bd1.reshape(1, -1),
                        Wd2, bd2.reshape(1, -1), n_graphs=n_graphs)


# speedup vs baseline: 1.6350x; 1.6350x over previous
import functools

import jax
import jax.numpy as jnp
import numpy as np
from jax import lax
from jax.experimental import pallas as pl
from jax.experimental.pallas import tpu as pltpu
from jax.experimental.pallas import tpu_sc as plsc

NC = 2
NS = 16
LANES = 16
DH = 128
CHUNK = 64


def _bf16_perm():
    a = np.empty(2 * DH // 2, np.int32)
    for j in range(2 * DH // 2):
        half, p = divmod(j, DH // 2)
        k, i = divmod(p, 16)
        a[j] = DH * half + 32 * k + i
    return np.concatenate([a, a + 16])


def _mp_sc(n_nodes: int, n_edges: int):
    ept = n_edges // NS
    nfull = ept // CHUNK
    ntrip = nfull // 3
    rest = ept - ntrip * 3 * CHUNK
    rpt_a = (-(-n_nodes // NS) + 7) // 8 * 8
    rpt_last = n_nodes - (NS - 1) * rpt_a

    mesh = plsc.VectorSubcoreMesh(
        core_axis_name="c", subcore_axis_name="s", num_cores=NC, num_subcores=NS
    )

    NB = 3
    scratch = (
        [pltpu.VMEM((CHUNK,), jnp.int32) for _ in range(NB)]
        + [pltpu.VMEM((CHUNK,), jnp.int32) for _ in range(NB)]
        + [pltpu.VMEM((CHUNK, DH), jnp.float32) for _ in range(NB)]
        + [pltpu.VMEM((CHUNK, DH // 2), jnp.int32) for _ in range(NB)]
        + [pltpu.VMEM_SHARED((n_nodes, DH), jnp.float32)]
        + [pltpu.SemaphoreType.DMA for _ in range(3 * NB)]
    )
    if rest:
        scratch += [pltpu.VMEM((rest,), jnp.int32),
                    pltpu.VMEM((rest,), jnp.int32)]

    @functools.partial(
        pl.kernel,
        out_type=jax.ShapeDtypeStruct((2 * n_nodes, DH), jnp.float32),
        mesh=mesh,
        scratch_types=scratch,
    )
    def mp(h2_hbm, e2_hbm, src_hbm, dst_hbm, agg_hbm, *scr):
        srcs = scr[0:NB]
        dsts = scr[NB:2 * NB]
        rows = scr[2 * NB:3 * NB]
        evs = scr[3 * NB:4 * NB]
        agg_sh = scr[4 * NB]
        semi = scr[4 * NB + 1:4 * NB + 1 + NB]
        semg = scr[4 * NB + 1 + NB:4 * NB + 1 + 2 * NB]
        sems = scr[4 * NB + 1 + 2 * NB:4 * NB + 1 + 3 * NB]
        if rest:
            src_t, dst_t = scr[-2], scr[-1]

        c = lax.axis_index("c")
        t = lax.axis_index("s")
        coff = c * n_nodes
        eoff = c * n_edges
        ebase = t * ept

        zeros = jnp.zeros((LANES,), jnp.float32)

        @pl.loop(0, CHUNK)
        def _zero_fill(i):
            for k in range(DH // LANES):
                rows[0][i, pl.ds(k * LANES, LANES)] = zeros

        r0 = t * rpt_a

        def _zero_rows(nrows):
            nf = nrows // CHUNK
            rem = nrows - nf * CHUNK

            @pl.loop(0, nf)
            def _z(i):
                pltpu.sync_copy(rows[0], agg_sh.at[pl.ds(r0 + i * CHUNK, CHUNK)])
            if rem:
                pltpu.sync_copy(rows[0].at[pl.ds(0, rem)],
                                agg_sh.at[pl.ds(r0 + nf * CHUNK, rem)])

        @pl.when(t < NS - 1)
        def _zero_main():
            _zero_rows(rpt_a)

        @pl.when(t == NS - 1)
        def _zero_last():
            _zero_rows(rpt_last)

        plsc.subcore_barrier()

        def issue_idx(chunk_i, b):
            base = ebase + chunk_i * CHUNK
            pltpu.async_copy(src_hbm.at[pl.ds(base, CHUNK)], srcs[b], semi[b])
            pltpu.async_copy(dst_hbm.at[pl.ds(base, CHUNK)], dsts[b], semi[b])

        def wait_idx(b):
            pltpu.make_async_copy(src_hbm.at[pl.ds(0, CHUNK)], srcs[b],
                                  semi[b]).wait()
            pltpu.make_async_copy(dst_hbm.at[pl.ds(0, CHUNK)], dsts[b],
                                  semi[b]).wait()

        def issue_gather(chunk_i, b):
            for k in range(CHUNK // LANES):
                sl = pl.ds(k * LANES, LANES)
                srcs[b][sl] = srcs[b][sl] + coff
            pltpu.async_copy(h2_hbm.at[srcs[b]], rows[b], semg[b])
            base = ebase + chunk_i * CHUNK
            pltpu.async_copy(e2_hbm.at[pl.ds(eoff + base, CHUNK)],
                             evs[b], semg[b])

        def wait_gather(b):
            pltpu.make_async_copy(h2_hbm.at[srcs[b]], rows[b], semg[b]).wait()
            pltpu.make_async_copy(e2_hbm.at[pl.ds(0, CHUNK)], evs[b],
                                  semg[b]).wait()

        def _combine_row(b, i):
            for k in range(DH // (2 * LANES)):
                w16 = evs[b][i, pl.ds(LANES * k, LANES)]
                ea = lax.bitcast_convert_type(lax.shift_left(w16, 16),
                                              jnp.float32)
                eb = lax.bitcast_convert_type(w16 & jnp.int32(-65536),
                                              jnp.float32)
                sla = pl.ds(2 * LANES * k, LANES)
                slb = pl.ds(2 * LANES * k + LANES, LANES)
                rows[b][i, sla] = jnp.maximum(rows[b][i, sla] + ea, 0.0)
                rows[b][i, slb] = jnp.maximum(rows[b][i, slb] + eb, 0.0)

        def relu_combine(b):
            @pl.loop(0, CHUNK)
            def _row(i):
                _combine_row(b, i)

        def issue_scatter(b):
            pltpu.async_copy(rows[b], agg_sh.at[dsts[b]], sems[b], add=True)

        def wait_scatter(b):
            pltpu.make_async_copy(rows[b], agg_sh.at[dsts[b]], sems[b]).wait()

        if ntrip > 0:
            issue_idx(0, 0)
            issue_idx(1, 1)
            issue_idx(2, 2)
            wait_idx(0)
            issue_gather(0, 0)
            wait_idx(1)
            issue_gather(1, 1)

            @pl.loop(0, ntrip)
            def _triple(tr):
                a = 3 * tr

                @pl.when(tr > 0)
                def _refill_c():
                    wait_scatter(2)
                    issue_idx(a + 2, 2)

                wait_idx(2)
                issue_gather(a + 2, 2)

                wait_gather(0)
                relu_combine(0)
                issue_scatter(0)
                wait_gather(1)
                relu_combine(1)
                issue_scatter(1)

                @pl.when(tr < ntrip - 1)
                def _refill_a():
                    wait_scatter(0)
                    issue_idx(a + 3, 0)

                wait_gather(2)
                relu_combine(2)
                issue_scatter(2)

                @pl.when(tr < ntrip - 1)
                def _refill_b():
                    wait_scatter(1)
                    issue_idx(a + 4, 1)

                @pl.when(tr < ntrip - 1)
                def _regather():
                    wait_idx(0)
                    issue_gather(a + 3, 0)
                    wait_idx(1)
                    issue_gather(a + 4, 1)

            wait_scatter(0)
            wait_scatter(1)
            wait_scatter(2)

        if rest:
            cc = rest
            base = ebase + ntrip * 3 * CHUNK
            pltpu.sync_copy(src_hbm.at[pl.ds(base, cc)], src_t)
            pltpu.sync_copy(dst_hbm.at[pl.ds(base, cc)], dst_t)
            for k in range(cc // LANES):
                sl = pl.ds(k * LANES, LANES)
                src_t[sl] = src_t[sl] + coff
            g = pltpu.async_copy(h2_hbm.at[src_t],
                                 rows[0].at[pl.ds(0, cc)], semg[0])
            le = pltpu.async_copy(e2_hbm.at[pl.ds(eoff + base, cc)],
                                  evs[0].at[pl.ds(0, cc)], semg[0])
            g.wait()
            le.wait()

            @pl.loop(0, cc)
            def _rowt(i):
                _combine_row(0, i)

            pltpu.sync_copy(rows[0].at[pl.ds(0, cc)],
                            agg_sh.at[dst_t], add=True)

        plsc.subcore_barrier()

        @pl.when(t < NS - 1)
        def _flush_main():
            pltpu.sync_copy(agg_sh.at[pl.ds(r0, rpt_a)],
                            agg_hbm.at[pl.ds(coff + r0, rpt_a)])

        @pl.when(t == NS - 1)
        def _flush_last():
            pltpu.sync_copy(agg_sh.at[pl.ds(r0, rpt_last)],
                            agg_hbm.at[pl.ds(coff + r0, rpt_last)])

    return mp


def _enc_h(x, w, b):
    n = x.shape[0]

    def body(x_ref, w_ref, b_ref, o_ref):
        h = jnp.dot(x_ref[...], w_ref[...], preferred_element_type=jnp.float32)
        h = h + b_ref[...]
        o_ref[0] = h[:, :DH]
        o_ref[1] = h[:, DH:]

    return pl.pallas_call(
        body,
        out_shape=jax.ShapeDtypeStruct((2, n, DH), jnp.float32),
    )(x, w, b)


def _enc_e(edge_attr, w, b, block_e=4000):
    e_cnt, dk = edge_attr.shape
    grid = (e_cnt // block_e,)

    def body(a_ref, w_ref, b_ref, o_ref):
        eb = jnp.dot(a_ref[...], w_ref[...], preferred_element_type=jnp.float32)
        eb = eb + b_ref[...]
        u = lax.bitcast_convert_type(eb, jnp.uint32) + jnp.uint32(0x8000)
        lo = u[:, :DH] >> 16
        hi = u[:, DH:] & jnp.uint32(0xFFFF0000)
        words = lax.bitcast_convert_type(hi | lo, jnp.int32)
        o_ref[0] = words[:, :DH // 2]
        o_ref[1] = words[:, DH // 2:]

    return pl.pallas_call(
        body,
        grid=grid,
        in_specs=[
            pl.BlockSpec((block_e, dk), lambda i: (i, 0)),
            pl.BlockSpec((dk, 2 * DH), lambda i: (0, 0)),
            pl.BlockSpec((1, 2 * DH), lambda i: (0, 0)),
        ],
        out_specs=pl.BlockSpec((2, block_e, DH // 2), lambda i: (0, i, 0)),
        out_shape=jax.ShapeDtypeStruct((2, e_cnt, DH // 2), jnp.int32),
    )(edge_attr, w, b)


def _layer_update(h2, agg2, w, b, gamma, beta):
    n = h2.shape[1]

    def body(h_ref, a_ref, w_ref, b_ref, g_ref, bt_ref, o_ref):
        s0 = h_ref[0] + a_ref[0]
        s1 = h_ref[1] + a_ref[1]
        z = jnp.dot(s0, w_ref[:DH], preferred_element_type=jnp.float32)
        z = z + jnp.dot(s1, w_ref[DH:], preferred_element_type=jnp.float32)
        z = z + b_ref[...]
        mean = jnp.mean(z, axis=0, keepdims=True)
        zc = z - mean
        var = jnp.mean(zc * zc, axis=0, keepdims=True)
        zn = zc / jnp.sqrt(var + 1e-5) * g_ref[...] + bt_ref[...]
        r = jnp.maximum(zn, 0.0)
        o_ref[0] = r[:, :DH] + h_ref[0]
        o_ref[1] = r[:, DH:] + h_ref[1]

    return pl.pallas_call(
        body,
        out_shape=jax.ShapeDtypeStruct((2, n, DH), jnp.float32),
    )(h2, agg2, w, b, gamma, beta)


def _pool_decode(h2, batch2, wd1, bd1, wd2, bd2, n_graphs=64):
    n = h2.shape[1]
    nout = wd2.shape[1]

    def body(h_ref, b_ref, w1_ref, b1_ref, w2_ref, b2_ref, o_ref):
        gids = lax.broadcasted_iota(jnp.int32, (n_graphs, n), 0)
        onehot = (gids == b_ref[...]).astype(jnp.float32)
        sums0 = jnp.dot(onehot, h_ref[0], preferred_element_type=jnp.float32)
        sums1 = jnp.dot(onehot, h_ref[1], preferred_element_type=jnp.float32)
        counts = jnp.sum(onehot, axis=1, keepdims=True)
        cmax = jnp.maximum(counts, 1.0)
        p0 = sums0 / cmax
        p1 = sums1 / cmax
        r = jnp.dot(p0, w1_ref[:DH], preferred_element_type=jnp.float32)
        r = r + jnp.dot(p1, w1_ref[DH:], preferred_element_type=jnp.float32)
        r = jnp.maximum(r + b1_ref[...], 0.0)
        o_ref[...] = jnp.dot(r, w2_ref[...], preferred_element_type=jnp.float32) + b2_ref[...]

    return pl.pallas_call(
        body,
        out_shape=jax.ShapeDtypeStruct((n_graphs, nout), jnp.float32),
    )(h2, batch2, wd1, bd1, wd2, bd2)


def kernel(x, edge_index, edge_attr, batch, W_in, b_in, W_e, b_e,
           Ws, bs, gammas, betas, Wd1, bd1, Wd2, bd2):
    n = x.shape[0]
    e_cnt = edge_index.shape[1]
    nlayer = Ws.shape[0]
    n_graphs = 64

    src = edge_index[0]
    dst = edge_index[1]

    h2 = _enc_h(x, W_in, b_in.reshape(1, -1))
    perm = _bf16_perm()
    e2 = _enc_e(edge_attr, W_e[:, perm], b_e[perm].reshape(1, -1))
    e2f = e2.reshape(2 * e_cnt, DH // 2)

    mp = _mp_sc(n, e_cnt)
    for l in range(nlayer):
        agg = mp(h2.reshape(2 * n, DH), e2f, src, dst)
        h2 = _layer_update(h2, agg.reshape(2, n, DH), Ws[l],
                           bs[l].reshape(1, -1), gammas[l].reshape(1, -1),
                           betas[l].reshape(1, -1))

    return _pool_decode(h2, batch.reshape(1, -1), Wd1, bd1.reshape(1, -1),
                        Wd2, bd2.reshape(1, -1), n_graphs=n_graphs)
